# Initial kernel scaffold; baseline (speedup 1.0000x reference)
import jax
import jax.numpy as jnp
from jax.experimental import pallas as pl


def kernel(x, W):
    raise NotImplementedError("write your pallas kernel here")


# speedup vs baseline: 1.8316x; 1.8316x over previous
import functools

import jax
import jax.numpy as jnp
from jax import lax
from jax.experimental import pallas as pl
from jax.experimental.pallas import tpu as pltpu
from jax.experimental.pallas import tpu_sc as plsc

VOCAB = 8192
BATCH = 4
SEQ = 2048
N_ROWS = BATCH * SEQ
NC, NS = 2, 16
NW = NC * NS
ROWS_PER_W = N_ROWS // NW
CHUNK = 8
N_CHUNKS = ROWS_PER_W // CHUNK


def _make_gather():
    mesh = plsc.VectorSubcoreMesh(core_axis_name="c", subcore_axis_name="s")

    @functools.partial(
        pl.kernel,
        out_type=jax.ShapeDtypeStruct((N_ROWS, VOCAB), jnp.float32),
        mesh=mesh,
        scratch_types=[
            pltpu.VMEM((ROWS_PER_W,), jnp.int32),
            pltpu.VMEM((CHUNK, VOCAB), jnp.float32),
            pltpu.SemaphoreType.DMA,
        ],
    )
    def gather_kernel(x_hbm, w_hbm, out_hbm, idx_v, rows_v, sem):
        wid = lax.axis_index("s") * NC + lax.axis_index("c")
        base = wid * ROWS_PER_W
        pltpu.sync_copy(x_hbm.at[pl.ds(base, ROWS_PER_W)], idx_v)

        def body(c, carry):
            off = c * CHUNK
            pltpu.async_copy(
                w_hbm.at[idx_v.at[pl.ds(off, CHUNK)]], rows_v, sem
            ).wait()
            pltpu.sync_copy(rows_v, out_hbm.at[pl.ds(base + off, CHUNK)])
            return carry

        lax.fori_loop(0, N_CHUNKS, body, 0)

    return gather_kernel


_gather = _make_gather()


def kernel(x, W):
    x_flat = x.reshape(N_ROWS).astype(jnp.int32)
    out = _gather(x_flat, W)
    return out.reshape(BATCH, SEQ, VOCAB)


# speedup vs baseline: 1.9586x; 1.0693x over previous
import functools

import jax
import jax.numpy as jnp
from jax import lax
from jax.experimental import pallas as pl
from jax.experimental.pallas import tpu as pltpu
from jax.experimental.pallas import tpu_sc as plsc

VOCAB = 8192
BATCH = 4
SEQ = 2048
N_ROWS = BATCH * SEQ
NC, NS = 2, 16
NW = NC * NS
ROWS_PER_W = N_ROWS // NW
CHUNK = 4
N_CHUNKS = ROWS_PER_W // CHUNK
NBUF = 2


def _make_gather():
    mesh = plsc.VectorSubcoreMesh(core_axis_name="c", subcore_axis_name="s")

    @functools.partial(
        pl.kernel,
        out_type=jax.ShapeDtypeStruct((N_ROWS, VOCAB), jnp.float32),
        mesh=mesh,
        scratch_types=[
            pltpu.VMEM((N_CHUNKS, CHUNK), jnp.int32),
            pltpu.VMEM((CHUNK, VOCAB), jnp.float32),
            pltpu.VMEM((CHUNK, VOCAB), jnp.float32),
            pltpu.SemaphoreType.DMA,
            pltpu.SemaphoreType.DMA,
            pltpu.SemaphoreType.DMA,
            pltpu.SemaphoreType.DMA,
        ],
    )
    def gather_kernel(x_hbm, w_hbm, out_hbm, idx_v, rows0, rows1,
                      gsem0, gsem1, osem0, osem1):
        rows = (rows0, rows1)
        gsem = (gsem0, gsem1)
        osem = (osem0, osem1)
        wid = lax.axis_index("s") * NC + lax.axis_index("c")
        base = wid * N_CHUNKS
        pltpu.sync_copy(x_hbm.at[pl.ds(base, N_CHUNKS)], idx_v)

        for b in range(NBUF):
            pltpu.async_copy(w_hbm.at[idx_v.at[b]], rows[b], gsem[b])

        def body(i, carry):
            g = i * NBUF
            for b in range(NBUF):
                c = g + b
                pltpu.make_async_copy(
                    w_hbm.at[idx_v.at[b]], rows[b], gsem[b]
                ).wait()
                pltpu.async_copy(
                    rows[b], out_hbm.at[pl.ds((base + c) * CHUNK, CHUNK)],
                    osem[b],
                ).wait()
                @pl.when(c + NBUF < N_CHUNKS)
                def _():
                    pltpu.async_copy(
                        w_hbm.at[idx_v.at[c + NBUF]], rows[b], gsem[b]
                    )
            return carry

        lax.fori_loop(0, N_CHUNKS // NBUF, body, 0)

    return gather_kernel


_gather = _make_gather()


def kernel(x, W):
    x2 = x.reshape(N_ROWS // CHUNK, CHUNK).astype(jnp.int32)
    out = _gather(x2, W)
    return out.reshape(BATCH, SEQ, VOCAB)


# speedup vs baseline: 1.9625x; 1.0020x over previous
import functools

import jax
import jax.numpy as jnp
from jax import lax
from jax.experimental import pallas as pl
from jax.experimental.pallas import tpu as pltpu
from jax.experimental.pallas import tpu_sc as plsc

VOCAB = 8192
BATCH = 4
SEQ = 2048
N_ROWS = BATCH * SEQ
NC, NS = 2, 16
NW = NC * NS
ROWS_PER_W = N_ROWS // NW
CHUNK = 4
N_CHUNKS = ROWS_PER_W // CHUNK
NBUF = 3


def _make_gather():
    mesh = plsc.VectorSubcoreMesh(core_axis_name="c", subcore_axis_name="s")

    @functools.partial(
        pl.kernel,
        out_type=jax.ShapeDtypeStruct((N_ROWS, VOCAB), jnp.float32),
        mesh=mesh,
        scratch_types=[
            pltpu.VMEM((N_CHUNKS, CHUNK), jnp.int32),
            pltpu.VMEM((CHUNK, VOCAB), jnp.float32),
            pltpu.VMEM((CHUNK, VOCAB), jnp.float32),
            pltpu.VMEM((CHUNK, VOCAB), jnp.float32),
            pltpu.SemaphoreType.DMA,
            pltpu.SemaphoreType.DMA,
            pltpu.SemaphoreType.DMA,
            pltpu.SemaphoreType.DMA,
            pltpu.SemaphoreType.DMA,
            pltpu.SemaphoreType.DMA,
        ],
    )
    def gather_kernel(x_hbm, w_hbm, out_hbm, idx_v, rows0, rows1, rows2,
                      gsem0, gsem1, gsem2, osem0, osem1, osem2):
        rows = (rows0, rows1, rows2)
        gsem = (gsem0, gsem1, gsem2)
        osem = (osem0, osem1, osem2)
        wid = lax.axis_index("s") * NC + lax.axis_index("c")
        base = wid * N_CHUNKS
        pltpu.sync_copy(x_hbm.at[pl.ds(base, N_CHUNKS)], idx_v)

        for b in range(NBUF - 1):
            pltpu.async_copy(w_hbm.at[idx_v.at[b]], rows[b], gsem[b])

        def body(i, carry):
            g = i * NBUF
            for b in range(NBUF):
                c = g + b
                bn = (b + 2) % NBUF
                pltpu.make_async_copy(
                    w_hbm.at[idx_v.at[b]], rows[b], gsem[b]
                ).wait()
                pltpu.async_copy(
                    rows[b], out_hbm.at[pl.ds((base + c) * CHUNK, CHUNK)],
                    osem[b],
                )
                @pl.when(c >= 1)
                def _():
                    pltpu.make_async_copy(
                        rows[bn],
                        out_hbm.at[pl.ds((base + c - 1) * CHUNK, CHUNK)],
                        osem[bn],
                    ).wait()
                @pl.when(c + 2 < N_CHUNKS)
                def _():
                    pltpu.async_copy(
                        w_hbm.at[idx_v.at[c + 2]], rows[bn], gsem[bn]
                    )
            return carry

        lax.fori_loop(0, N_CHUNKS // NBUF, body, 0)
        c_last = N_CHUNKS - 1
        b_last = c_last % NBUF
        b_prev = (c_last - 1) % NBUF
        pltpu.make_async_copy(
            w_hbm.at[idx_v.at[b_last]], rows[b_last], gsem[b_last]
        ).wait()
        pltpu.async_copy(
            rows[b_last],
            out_hbm.at[pl.ds((base + c_last) * CHUNK, CHUNK)],
            osem[b_last],
        )
        pltpu.make_async_copy(
            rows[b_prev],
            out_hbm.at[pl.ds((base + c_last - 1) * CHUNK, CHUNK)],
            osem[b_prev],
        ).wait()
        pltpu.make_async_copy(
            rows[b_last],
            out_hbm.at[pl.ds((base + c_last) * CHUNK, CHUNK)],
            osem[b_last],
        ).wait()

    return gather_kernel


_gather = _make_gather()


def kernel(x, W):
    x2 = x.reshape(N_ROWS // CHUNK, CHUNK).astype(jnp.int32)
    out = _gather(x2, W)
    return out.reshape(BATCH, SEQ, VOCAB)
